# baseline (speedup 1.0000x reference)
import jax
import jax.numpy as jnp
from jax import lax
from jax.experimental import pallas as pl
from jax.experimental.pallas import tpu as pltpu
from jax.experimental.pallas import tpu_sc as plsc

N_CLASSES = 1000
IMAGES_PER_CLASS = 10
IM_DIM = 32
IM_CHAN = 3
BATCH = 1024
ROW = IM_DIM * IM_DIM * IM_CHAN

_INFO = plsc.get_sparse_core_info()
_NC = _INFO.num_cores
_NS = _INFO.num_subcores
_NW = _NC * _NS
_BPW = BATCH // _NW
_LANES = _INFO.num_lanes


def _body(noise_hbm, labels_hbm, table_hbm, out_hbm, lab_v, noi_v, idx_v,
          rows_v, sem):
  wid = lax.axis_index("s") * _NC + lax.axis_index("c")
  base = wid * _BPW
  pltpu.sync_copy(labels_hbm.at[pl.ds(base, _BPW)], lab_v)
  pltpu.sync_copy(noise_hbm.at[pl.ds(base, _BPW)], noi_v)
  for j in range(_BPW // _LANES):
    sl = pl.ds(j * _LANES, _LANES)
    idx_v[sl] = lab_v[sl] * IMAGES_PER_CLASS + noi_v[sl]
  pltpu.async_copy(table_hbm.at[idx_v], rows_v, sem).wait()
  pltpu.sync_copy(rows_v, out_hbm.at[pl.ds(base, _BPW)])


@jax.jit
def _gather(noise, labels, table):
  mesh = plsc.VectorSubcoreMesh(core_axis_name="c", subcore_axis_name="s")
  return pl.kernel(
      _body,
      out_type=jax.ShapeDtypeStruct((BATCH, ROW // 128, 128), jnp.float32),
      mesh=mesh,
      scratch_types=[
          pltpu.VMEM((_BPW,), jnp.int32),
          pltpu.VMEM((_BPW,), jnp.int32),
          pltpu.VMEM((_BPW,), jnp.int32),
          pltpu.VMEM((_BPW, ROW // 128, 128), jnp.float32),
          pltpu.SemaphoreType.DMA,
      ],
  )(noise, labels, table)


def kernel(noise, labels, batches, is_training, data):
  table = data.reshape(N_CLASSES * IMAGES_PER_CLASS, ROW // 128, 128)
  out = _gather(noise, labels, table)
  return out.reshape(BATCH, IM_DIM, IM_DIM, IM_CHAN)


# speedup vs baseline: 10.6827x; 10.6827x over previous
import jax
import jax.numpy as jnp
from jax import lax
from jax.experimental import pallas as pl
from jax.experimental.pallas import tpu as pltpu
from jax.experimental.pallas import tpu_sc as plsc

N_CLASSES = 1000
IMAGES_PER_CLASS = 10
IM_DIM = 32
IM_CHAN = 3
BATCH = 1024

_INFO = plsc.get_sparse_core_info()
_NC = _INFO.num_cores
_NS = _INFO.num_subcores
_NW = _NC * _NS
_LANES = _INFO.num_lanes

_PAIRS = IM_DIM * IM_CHAN
_PPW = _PAIRS // _NW
_WG = 8
_NSTEPS = _PPW * (IM_DIM // _WG)
_ROWS = IM_DIM * _PAIRS
_SLAB = _PAIRS * IM_DIM


def _body(noise_hbm, labels_hbm, table_hbm, out_hbm, noi_v, lab_v, b_v,
          obuf_v, sem):
  wid = lax.axis_index("s") * _NC + lax.axis_index("c")
  pltpu.sync_copy(noise_hbm, noi_v)
  pltpu.sync_copy(labels_hbm, lab_v)

  def step(s, carry):
    p = wid * _PPW + s // (IM_DIM // _WG)
    g = s % (IM_DIM // _WG)
    rbase = pl.multiple_of(p * IM_DIM + g * _WG, _WG)
    cps = []
    for n in range(IMAGES_PER_CLASS):
      row = pl.multiple_of(n * _SLAB + rbase, _WG)
      cps.append(pltpu.async_copy(table_hbm.at[pl.ds(row, _WG)], b_v.at[n],
                                  sem))
    for cp in cps:
      cp.wait()
    for w in range(_WG):
      def gather16(j, c, w=w):
        nj = noi_v[pl.ds(j * _LANES, _LANES)]
        lj = lab_v[pl.ds(j * _LANES, _LANES)]
        wv = jnp.full((_LANES,), w, jnp.int32)
        obuf_v[w, pl.ds(j * _LANES, _LANES)] = plsc.load_gather(
            b_v, [nj, wv, lj])
        return c
      lax.fori_loop(0, BATCH // _LANES, gather16, 0)
    pltpu.sync_copy(obuf_v, out_hbm.at[pl.ds(rbase, _WG)])
    return carry

  lax.fori_loop(0, _NSTEPS, step, 0)


@jax.jit
def _gather(noise, labels, table):
  mesh = plsc.VectorSubcoreMesh(core_axis_name="c", subcore_axis_name="s")
  return pl.kernel(
      _body,
      out_type=jax.ShapeDtypeStruct((_ROWS, BATCH), jnp.float32),
      mesh=mesh,
      compiler_params=pltpu.CompilerParams(needs_layout_passes=False),
      scratch_types=[
          pltpu.VMEM((BATCH,), jnp.int32),
          pltpu.VMEM((BATCH,), jnp.int32),
          pltpu.VMEM((IMAGES_PER_CLASS, _WG, N_CLASSES), jnp.float32),
          pltpu.VMEM((_WG, BATCH), jnp.float32),
          pltpu.SemaphoreType.DMA,
      ],
  )(noise, labels, table)


def kernel(noise, labels, batches, is_training, data):
  table = data.transpose(1, 2, 4, 3, 0).reshape(_SLAB * IMAGES_PER_CLASS,
                                                N_CLASSES)
  buf = _gather(noise, labels, table)
  return buf.reshape(IM_DIM, IM_CHAN, IM_DIM, BATCH).transpose(3, 0, 2, 1)


# speedup vs baseline: 15.3042x; 1.4326x over previous
import jax
import jax.numpy as jnp
from jax import lax
from jax.experimental import pallas as pl
from jax.experimental.pallas import tpu as pltpu
from jax.experimental.pallas import tpu_sc as plsc

N_CLASSES = 1000
IMAGES_PER_CLASS = 10
IM_DIM = 32
IM_CHAN = 3
BATCH = 1024

_INFO = plsc.get_sparse_core_info()
_NC = _INFO.num_cores
_NS = _INFO.num_subcores
_NW = _NC * _NS
_LANES = _INFO.num_lanes

_PAIRS = IM_DIM * IM_CHAN
_PPW = _PAIRS // _NW
_WG = 8
_NSTEPS = _PPW * (IM_DIM // _WG)
_ROWS = IM_DIM * _PAIRS
_SLAB = _PAIRS * IM_DIM


_HALF = IMAGES_PER_CLASS // 2


def _body(noise_hbm, labels_hbm, table_hbm, out_hbm, noi_v, lab_v, b0_v,
          b1_v, obuf_v, sem_a, sem_b):
  wid = lax.axis_index("s") * _NC + lax.axis_index("c")
  pltpu.sync_copy(noise_hbm, noi_v)
  pltpu.sync_copy(labels_hbm, lab_v)

  def rbase_of(s):
    p = wid * _PPW + s // (IM_DIM // _WG)
    g = s % (IM_DIM // _WG)
    return pl.multiple_of(p * IM_DIM + g * _WG, _WG)

  def fire(s, half, buf, sem):
    rbase = rbase_of(s)
    for k in range(_HALF):
      n = half * _HALF + k
      row = pl.multiple_of(n * _SLAB + rbase, _WG)
      pltpu.async_copy(table_hbm.at[pl.ds(row, _WG)], buf.at[k], sem)

  def drain(s, half, buf, sem):
    rbase = rbase_of(s)
    for k in range(_HALF):
      n = half * _HALF + k
      row = pl.multiple_of(n * _SLAB + rbase, _WG)
      pltpu.make_async_copy(table_hbm.at[pl.ds(row, _WG)], buf.at[k],
                            sem).wait()

  fire(0, 0, b0_v, sem_a)
  fire(0, 1, b1_v, sem_b)

  def step(t, carry):
    rbase = rbase_of(t)
    drain(t, 0, b0_v, sem_a)

    def gather_a(j, c):
      nj = jnp.minimum(noi_v[pl.ds(j * _LANES, _LANES)], _HALF - 1)
      lj = lab_v[pl.ds(j * _LANES, _LANES)]
      for w in range(_WG):
        wv = jnp.full((_LANES,), w, jnp.int32)
        obuf_v[w, pl.ds(j * _LANES, _LANES)] = plsc.load_gather(
            b0_v, [nj, wv, lj])
      return c
    lax.fori_loop(0, BATCH // _LANES, gather_a, 0)

    @pl.when(t + 1 < _NSTEPS)
    def _():
      fire(t + 1, 0, b0_v, sem_a)

    drain(t, 1, b1_v, sem_b)

    def gather_b(j, c):
      nj0 = noi_v[pl.ds(j * _LANES, _LANES)]
      hi = nj0 >= _HALF
      nj = jnp.maximum(nj0 - _HALF, 0)
      lj = lab_v[pl.ds(j * _LANES, _LANES)]
      for w in range(_WG):
        wv = jnp.full((_LANES,), w, jnp.int32)
        g = plsc.load_gather(b1_v, [nj, wv, lj])
        sl = pl.ds(j * _LANES, _LANES)
        obuf_v[w, sl] = jnp.where(hi, g, obuf_v[w, sl])
      return c
    lax.fori_loop(0, BATCH // _LANES, gather_b, 0)

    @pl.when(t + 1 < _NSTEPS)
    def _():
      fire(t + 1, 1, b1_v, sem_b)

    pltpu.sync_copy(obuf_v, out_hbm.at[pl.ds(rbase, _WG)])
    return carry

  lax.fori_loop(0, _NSTEPS, step, 0)


@jax.jit
def _gather(noise, labels, table):
  mesh = plsc.VectorSubcoreMesh(core_axis_name="c", subcore_axis_name="s")
  return pl.kernel(
      _body,
      out_type=jax.ShapeDtypeStruct((_ROWS, BATCH), jnp.float32),
      mesh=mesh,
      compiler_params=pltpu.CompilerParams(needs_layout_passes=False),
      scratch_types=[
          pltpu.VMEM((BATCH,), jnp.int32),
          pltpu.VMEM((BATCH,), jnp.int32),
          pltpu.VMEM((_HALF, _WG, N_CLASSES), jnp.float32),
          pltpu.VMEM((_HALF, _WG, N_CLASSES), jnp.float32),
          pltpu.VMEM((_WG, BATCH), jnp.float32),
          pltpu.SemaphoreType.DMA,
          pltpu.SemaphoreType.DMA,
      ],
  )(noise, labels, table)


def kernel(noise, labels, batches, is_training, data):
  table = data.transpose(1, 2, 4, 3, 0).reshape(_SLAB * IMAGES_PER_CLASS,
                                                N_CLASSES)
  buf = _gather(noise, labels, table)
  return buf.reshape(IM_DIM, IM_CHAN, IM_DIM, BATCH).transpose(3, 0, 2, 1)


# speedup vs baseline: 17.0535x; 1.1143x over previous
import jax
import jax.numpy as jnp
from jax import lax
from jax.experimental import pallas as pl
from jax.experimental.pallas import tpu as pltpu
from jax.experimental.pallas import tpu_sc as plsc

N_CLASSES = 1000
IMAGES_PER_CLASS = 10
IM_DIM = 32
IM_CHAN = 3
BATCH = 1024

_INFO = plsc.get_sparse_core_info()
_NC = _INFO.num_cores
_NS = _INFO.num_subcores
_NW = _NC * _NS
_LANES = _INFO.num_lanes

_PAIRS = IM_DIM * IM_CHAN
_PPW = _PAIRS // _NW
_WG = 8
_NSTEPS = _PPW * (IM_DIM // _WG)
_ROWS = IM_DIM * _PAIRS
_SLAB = _PAIRS * IM_DIM


_HALF = IMAGES_PER_CLASS // 2


def _body(noise_hbm, labels_hbm, table_hbm, out_hbm, noi_v, lab_v, b0_v,
          b1_v, o0_v, o1_v, sem_a, sem_b, sem_o):
  wid = lax.axis_index("s") * _NC + lax.axis_index("c")
  pltpu.sync_copy(noise_hbm, noi_v)
  pltpu.sync_copy(labels_hbm, lab_v)
  obufs = (o0_v, o1_v)

  def rbase_of(s):
    p = wid * _PPW + s // (IM_DIM // _WG)
    g = s % (IM_DIM // _WG)
    return pl.multiple_of(p * IM_DIM + g * _WG, _WG)

  def fire(s, half, buf, sem):
    rbase = rbase_of(s)
    for k in range(_HALF):
      n = half * _HALF + k
      row = pl.multiple_of(n * _SLAB + rbase, _WG)
      pltpu.async_copy(table_hbm.at[pl.ds(row, _WG)], buf.at[k], sem)

  def drain(s, half, buf, sem):
    rbase = rbase_of(s)
    for k in range(_HALF):
      n = half * _HALF + k
      row = pl.multiple_of(n * _SLAB + rbase, _WG)
      pltpu.make_async_copy(table_hbm.at[pl.ds(row, _WG)], buf.at[k],
                            sem).wait()

  def stage(t, obuf, last):
    rbase = rbase_of(t)
    drain(t, 0, b0_v, sem_a)

    def gather_a(j, c):
      nj = jnp.minimum(noi_v[pl.ds(j * _LANES, _LANES)], _HALF - 1)
      lj = lab_v[pl.ds(j * _LANES, _LANES)]
      for w in range(_WG):
        wv = jnp.full((_LANES,), w, jnp.int32)
        obuf[w, pl.ds(j * _LANES, _LANES)] = plsc.load_gather(
            b0_v, [nj, wv, lj])
      return c
    lax.fori_loop(0, BATCH // _LANES, gather_a, 0)

    if last is None:
      fire(t + 1, 0, b0_v, sem_a)
    else:
      @pl.when(last)
      def _():
        fire(t + 1, 0, b0_v, sem_a)

    drain(t, 1, b1_v, sem_b)

    def gather_b(j, c):
      nj0 = noi_v[pl.ds(j * _LANES, _LANES)]
      hi = nj0 >= _HALF
      nj = jnp.maximum(nj0 - _HALF, 0)
      lj = lab_v[pl.ds(j * _LANES, _LANES)]
      lane = j * _LANES + lax.iota(jnp.int32, _LANES)
      for w in range(_WG):
        wv = jnp.full((_LANES,), w, jnp.int32)
        g = plsc.load_gather(b1_v, [nj, wv, lj])
        plsc.store_scatter(obuf, [wv, lane], g, mask=hi)
      return c
    lax.fori_loop(0, BATCH // _LANES, gather_b, 0)

    if last is None:
      fire(t + 1, 1, b1_v, sem_b)
    else:
      @pl.when(last)
      def _():
        fire(t + 1, 1, b1_v, sem_b)

    pltpu.async_copy(obuf, out_hbm.at[pl.ds(rbase, _WG)], sem_o)

  fire(0, 0, b0_v, sem_a)
  fire(0, 1, b1_v, sem_b)

  def pair(u, carry):
    for par in range(2):
      t = 2 * u + par

      @pl.when(u >= 1)
      def _():
        tp = t - 2
        pltpu.make_async_copy(
            obufs[par], out_hbm.at[pl.ds(rbase_of(tp), _WG)], sem_o).wait()

      stage(t, obufs[par],
            None if par == 0 else (u < _NSTEPS // 2 - 1))
    return carry

  lax.fori_loop(0, _NSTEPS // 2, pair, 0)
  for par in range(2):
    tp = _NSTEPS - 2 + par
    pltpu.make_async_copy(obufs[par],
                          out_hbm.at[pl.ds(rbase_of(tp), _WG)], sem_o).wait()


@jax.jit
def _gather(noise, labels, table):
  mesh = plsc.VectorSubcoreMesh(core_axis_name="c", subcore_axis_name="s")
  return pl.kernel(
      _body,
      out_type=jax.ShapeDtypeStruct((_ROWS, BATCH), jnp.float32),
      mesh=mesh,
      compiler_params=pltpu.CompilerParams(needs_layout_passes=False),
      scratch_types=[
          pltpu.VMEM((BATCH,), jnp.int32),
          pltpu.VMEM((BATCH,), jnp.int32),
          pltpu.VMEM((_HALF, _WG, N_CLASSES), jnp.float32),
          pltpu.VMEM((_HALF, _WG, N_CLASSES), jnp.float32),
          pltpu.VMEM((_WG, BATCH), jnp.float32),
          pltpu.VMEM((_WG, BATCH), jnp.float32),
          pltpu.SemaphoreType.DMA,
          pltpu.SemaphoreType.DMA,
          pltpu.SemaphoreType.DMA,
      ],
  )(noise, labels, table)


def kernel(noise, labels, batches, is_training, data):
  table = data.transpose(1, 2, 4, 3, 0).reshape(_SLAB * IMAGES_PER_CLASS,
                                                N_CLASSES)
  buf = _gather(noise, labels, table)
  return buf.reshape(IM_DIM, IM_CHAN, IM_DIM, BATCH).transpose(3, 0, 2, 1)


# speedup vs baseline: 17.4164x; 1.0213x over previous
import jax
import jax.numpy as jnp
from jax import lax
from jax.experimental import pallas as pl
from jax.experimental.pallas import tpu as pltpu
from jax.experimental.pallas import tpu_sc as plsc

N_CLASSES = 1000
IMAGES_PER_CLASS = 10
IM_DIM = 32
IM_CHAN = 3
BATCH = 1024

_INFO = plsc.get_sparse_core_info()
_NC = _INFO.num_cores
_NS = _INFO.num_subcores
_NW = _NC * _NS
_LANES = _INFO.num_lanes

_PAIRS = IM_DIM * IM_CHAN
_PPW = _PAIRS // _NW
_WG = 8
_NSTEPS = _PPW * (IM_DIM // _WG)
_ROWS = IM_DIM * _PAIRS
_SLAB = _PAIRS * IM_DIM


_HALF = IMAGES_PER_CLASS // 2


def _body(noise_hbm, labels_hbm, table_hbm, out_hbm, noi_v, lab_v, b0_v,
          b1_v, o0_v, o1_v, sem_a, sem_b, sem_o):
  wid = lax.axis_index("s") * _NC + lax.axis_index("c")
  obufs = (o0_v, o1_v)

  def rbase_of(s):
    p = wid * _PPW + s // (IM_DIM // _WG)
    g = s % (IM_DIM // _WG)
    return pl.multiple_of(p * IM_DIM + g * _WG, _WG)

  def fire(s, half, buf, sem):
    rbase = rbase_of(s)
    for k in range(_HALF):
      n = half * _HALF + k
      row = pl.multiple_of(n * _SLAB + rbase, _WG)
      pltpu.async_copy(table_hbm.at[pl.ds(row, _WG)], buf.at[k], sem)

  def drain(s, half, buf, sem):
    rbase = rbase_of(s)
    for k in range(_HALF):
      n = half * _HALF + k
      row = pl.multiple_of(n * _SLAB + rbase, _WG)
      pltpu.make_async_copy(table_hbm.at[pl.ds(row, _WG)], buf.at[k],
                            sem).wait()

  def stage(t, obuf, last):
    rbase = rbase_of(t)
    drain(t, 0, b0_v, sem_a)

    def gather_a(j, c):
      nj = jnp.minimum(noi_v[pl.ds(j * _LANES, _LANES)], _HALF - 1)
      lj = lab_v[pl.ds(j * _LANES, _LANES)]
      for w in range(_WG):
        wv = jnp.full((_LANES,), w, jnp.int32)
        obuf[w, pl.ds(j * _LANES, _LANES)] = plsc.load_gather(
            b0_v, [nj, wv, lj])
      return c
    lax.fori_loop(0, BATCH // _LANES, gather_a, 0)

    if last is None:
      fire(t + 1, 0, b0_v, sem_a)
    else:
      @pl.when(last)
      def _():
        fire(t + 1, 0, b0_v, sem_a)

    drain(t, 1, b1_v, sem_b)

    def gather_b(j, c):
      nj0 = noi_v[pl.ds(j * _LANES, _LANES)]
      hi = nj0 >= _HALF
      nj = jnp.maximum(nj0 - _HALF, 0)
      lj = lab_v[pl.ds(j * _LANES, _LANES)]
      lane = j * _LANES + lax.iota(jnp.int32, _LANES)
      for w in range(_WG):
        wv = jnp.full((_LANES,), w, jnp.int32)
        g = plsc.load_gather(b1_v, [nj, wv, lj])
        plsc.store_scatter(obuf, [wv, lane], g, mask=hi)
      return c
    lax.fori_loop(0, BATCH // _LANES, gather_b, 0)

    if last is None:
      fire(t + 1, 1, b1_v, sem_b)
    else:
      @pl.when(last)
      def _():
        fire(t + 1, 1, b1_v, sem_b)

    pltpu.async_copy(obuf, out_hbm.at[pl.ds(rbase, _WG)], sem_o)

  fire(0, 0, b0_v, sem_a)
  fire(0, 1, b1_v, sem_b)
  pltpu.sync_copy(noise_hbm, noi_v)
  pltpu.sync_copy(labels_hbm, lab_v)

  def pair(u, carry):
    for par in range(2):
      t = 2 * u + par

      @pl.when(u >= 1)
      def _():
        tp = t - 2
        pltpu.make_async_copy(
            obufs[par], out_hbm.at[pl.ds(rbase_of(tp), _WG)], sem_o).wait()

      stage(t, obufs[par],
            None if par == 0 else (u < _NSTEPS // 2 - 1))
    return carry

  lax.fori_loop(0, _NSTEPS // 2, pair, 0)
  for par in range(2):
    tp = _NSTEPS - 2 + par
    pltpu.make_async_copy(obufs[par],
                          out_hbm.at[pl.ds(rbase_of(tp), _WG)], sem_o).wait()


@jax.jit
def _gather(noise, labels, table):
  mesh = plsc.VectorSubcoreMesh(core_axis_name="c", subcore_axis_name="s")
  return pl.kernel(
      _body,
      out_type=jax.ShapeDtypeStruct((_ROWS, BATCH), jnp.float32),
      mesh=mesh,
      compiler_params=pltpu.CompilerParams(needs_layout_passes=False),
      scratch_types=[
          pltpu.VMEM((BATCH,), jnp.int32),
          pltpu.VMEM((BATCH,), jnp.int32),
          pltpu.VMEM((_HALF, _WG, N_CLASSES), jnp.float32),
          pltpu.VMEM((_HALF, _WG, N_CLASSES), jnp.float32),
          pltpu.VMEM((_WG, BATCH), jnp.float32),
          pltpu.VMEM((_WG, BATCH), jnp.float32),
          pltpu.SemaphoreType.DMA,
          pltpu.SemaphoreType.DMA,
          pltpu.SemaphoreType.DMA,
      ],
  )(noise, labels, table)


def kernel(noise, labels, batches, is_training, data):
  table = data.transpose(1, 2, 4, 3, 0).reshape(_SLAB * IMAGES_PER_CLASS,
                                                N_CLASSES)
  buf = _gather(noise, labels, table)
  return buf.reshape(IM_DIM, IM_CHAN, IM_DIM, BATCH).transpose(3, 0, 2, 1)
